# baseline (speedup 1.0000x reference)
import functools

import jax
import jax.numpy as jnp
from jax import lax
from jax.experimental import pallas as pl
from jax.experimental.pallas import tpu as pltpu
from jax.experimental.pallas import tpu_sc as plsc

BATCH = 16384
EMB = 32
N_USERS = 1000000
N_TOURS = 100000

_info = plsc.get_sparse_core_info()
_NC = _info.num_cores
_NS = _info.num_subcores
_L = _info.num_lanes
_NW = _NC * _NS
_BPW = BATCH // _NW

_UCHUNK = 131072
_TCHUNK = 51200


def _score_body(tT_ref, w_ref, b_ref, out_ref):
    out_ref[...] = jnp.sum(tT_ref[...] * w_ref[...], axis=0) + b_ref[0]


_SCHUNK = 65536


def _score2_body(a_ref, b2_ref, w_ref, b_ref, out_ref):
    out_ref[pl.ds(0, _SCHUNK)] = (
        jnp.sum(a_ref[...] * w_ref[...], axis=0) + b_ref[0])
    out_ref[pl.ds(_SCHUNK, _SCHUNK)] = (
        jnp.sum(b2_ref[...] * w_ref[...], axis=0) + b_ref[0])


def _scores2(tT, wcol, bias, n):
    grid = (n + 2 * _SCHUNK - 1) // (2 * _SCHUNK)
    return pl.pallas_call(
        _score2_body,
        grid=(grid,),
        in_specs=[
            pl.BlockSpec((EMB, _SCHUNK), lambda i: (0, 2 * i)),
            pl.BlockSpec((EMB, _SCHUNK), lambda i: (0, 2 * i + 1)),
            pl.BlockSpec((EMB, 1), lambda i: (0, 0)),
            pl.BlockSpec(memory_space=pltpu.SMEM),
        ],
        out_specs=pl.BlockSpec((2 * _SCHUNK,), lambda i: (i,)),
        out_shape=jax.ShapeDtypeStruct((n,), jnp.float32),
    )(tT, tT, wcol, bias)


def _scores(tT, wcol, bias, n, chunk):
    grid = (n + chunk - 1) // chunk
    return pl.pallas_call(
        _score_body,
        grid=(grid,),
        in_specs=[
            pl.BlockSpec((EMB, chunk), lambda i: (0, i)),
            pl.BlockSpec((EMB, 1), lambda i: (0, 0)),
            pl.BlockSpec(memory_space=pltpu.SMEM),
        ],
        out_specs=pl.BlockSpec((chunk,), lambda i: (i,)),
        out_shape=jax.ShapeDtypeStruct((n,), jnp.float32),
    )(tT, wcol, bias)


def _gather_body(users_hbm, tours_hbm, us_hbm, ts_hbm, out_hbm,
                 uidx, tidx, uval, tval, outv, sem_u, sem_t):
    wid = lax.axis_index("s") * _NC + lax.axis_index("c")
    base = wid * _BPW
    pltpu.sync_copy(users_hbm.at[pl.ds(base, _BPW)], uidx)
    pltpu.sync_copy(tours_hbm.at[pl.ds(base, _BPW)], tidx)
    cu = pltpu.async_copy(us_hbm.at[uidx], uval, sem_u)
    ct = pltpu.async_copy(ts_hbm.at[tidx], tval, sem_t)
    cu.wait()
    ct.wait()

    def group(g, carry):
        sl = pl.ds(g * _L, _L)
        outv[sl] = uval[sl] + tval[sl]
        return carry

    lax.fori_loop(0, _BPW // _L, group, 0)
    pltpu.sync_copy(outv, out_hbm.at[pl.ds(base, _BPW)])


@jax.jit
def kernel(users, tours, user_table, tour_table, W, b):
    wu = W[0, :EMB].reshape(EMB, 1)
    wt = W[0, EMB:].reshape(EMB, 1)
    zero = jnp.zeros((1,), jnp.float32)
    uscore = _scores2(user_table.T, wu, zero, N_USERS)
    tscore = _scores(tour_table.T, wt, b, N_TOURS, _TCHUNK)

    run = pl.kernel(
        _gather_body,
        out_type=jax.ShapeDtypeStruct((BATCH,), jnp.float32),
        mesh=plsc.VectorSubcoreMesh(core_axis_name="c", subcore_axis_name="s"),
        compiler_params=pltpu.CompilerParams(
            needs_layout_passes=False, use_tc_tiling_on_sc=False),
        scratch_types=[
            pltpu.VMEM((_BPW,), jnp.int32),
            pltpu.VMEM((_BPW,), jnp.int32),
            pltpu.VMEM((_BPW,), jnp.float32),
            pltpu.VMEM((_BPW,), jnp.float32),
            pltpu.VMEM((_BPW,), jnp.float32),
            pltpu.SemaphoreType.DMA,
            pltpu.SemaphoreType.DMA,
        ],
    )
    out = run(users.astype(jnp.int32), tours.astype(jnp.int32), uscore, tscore)
    return out.reshape(BATCH, 1)
